# baseline (speedup 1.0000x reference)
import functools
import jax
import jax.numpy as jnp
from jax import lax
from jax.experimental import pallas as pl
from jax.experimental.pallas import tpu as pltpu
from jax.experimental.pallas import tpu_sc as plsc

_NW = 32
_RPW = 312
_RLAST = 10000 - 31 * _RPW


def _sc_body(x_hbm, out_hbm, xs_v, acc_v, sem):
    wid = lax.axis_index("s") * 2 + lax.axis_index("c")
    base = wid * _RPW

    def do_rows(nrows):
        pltpu.sync_copy(x_hbm.at[pl.ds(base, nrows), :], xs_v.at[pl.ds(0, nrows), :])

        def body(i, accs):
            return tuple(
                accs[k] + xs_v[i, pl.ds(k * 16, 16)] for k in range(8)
            )

        accs = lax.fori_loop(
            0, nrows, body, tuple(jnp.zeros((16,), jnp.float32) for _ in range(8))
        )
        for k in range(8):
            acc_v[0, pl.ds(k * 16, 16)] = accs[k]
        pltpu.sync_copy(acc_v, out_hbm.at[pl.ds(wid, 1), :])

    @pl.when(wid < _NW - 1)
    def _main():
        do_rows(_RPW)

    @pl.when(wid == _NW - 1)
    def _tail():
        do_rows(_RLAST)


def _sc_partials(x):
    kfn = functools.partial(
        pl.kernel,
        mesh=plsc.VectorSubcoreMesh(core_axis_name="c", subcore_axis_name="s"),
        out_type=jax.ShapeDtypeStruct((_NW, 128), jnp.float32),
        scratch_types=[
            pltpu.VMEM((_RLAST, 128), jnp.float32),
            pltpu.VMEM((1, 128), jnp.float32),
            pltpu.SemaphoreType.DMA,
        ],
    )
    return kfn(_sc_body)(x)


def _tc_body(n, x_ref, w_ref, p_ref, xout_ref, score_ref):
    xv = x_ref[...]
    w = w_ref[...]
    d = w.shape[1]
    w2 = w * lax.rsqrt(jnp.sum(w * w))
    colsum = jnp.sum(p_ref[...], axis=0, keepdims=True)
    c = jnp.sum(colsum * w2) / n
    s_row = lax.dot_general(
        w2, xv, (((1,), (1,)), ((), ())), preferred_element_type=jnp.float32
    )
    score_ref[...] = jnp.tanh(s_row - c)
    w2t = lax.transpose(w2, (1, 0))
    wb = lax.broadcast_in_dim(w2t, (d, d), (0, 1))
    sb = lax.dot_general(
        xv, wb, (((1,), (0,)), ((), ())), preferred_element_type=jnp.float32
    )
    xout_ref[...] = xv * jnp.tanh(sb - c)


def kernel(x, edge_index, weight):
    n, d = x.shape
    partials = _sc_partials(x)

    def body(*refs):
        _tc_body(n, *refs)

    x_out, score = pl.pallas_call(
        body,
        out_shape=(
            jax.ShapeDtypeStruct((n, d), x.dtype),
            jax.ShapeDtypeStruct((1, n), x.dtype),
        ),
    )(x, weight, partials)
    return x_out, score


# speedup vs baseline: 4.1389x; 4.1389x over previous
import jax
import jax.numpy as jnp
from jax import lax
from jax.experimental import pallas as pl


def _body(n, x_ref, w_ref, xout_ref, score_ref):
    xv = x_ref[...]
    w = w_ref[...]
    d = w.shape[1]
    w2 = w * lax.rsqrt(jnp.sum(w * w))
    s_row = lax.dot_general(
        w2, xv, (((1,), (1,)), ((), ())), preferred_element_type=jnp.float32
    )
    c = jnp.sum(s_row) / n
    score_ref[...] = jnp.tanh(s_row - c)
    w2t = lax.transpose(w2, (1, 0))
    wb = lax.broadcast_in_dim(w2t, (d, d), (0, 1))
    sb = lax.dot_general(
        xv, wb, (((1,), (0,)), ((), ())), preferred_element_type=jnp.float32
    )
    xout_ref[...] = xv * jnp.tanh(sb - c)


def kernel(x, edge_index, weight):
    n, d = x.shape

    def body(*refs):
        _body(n, *refs)

    x_out, score = pl.pallas_call(
        body,
        out_shape=(
            jax.ShapeDtypeStruct((n, d), x.dtype),
            jax.ShapeDtypeStruct((1, n), x.dtype),
        ),
    )(x, weight)
    return x_out, score
